# baseline (speedup 1.0000x reference)
import functools

import jax
import jax.numpy as jnp
from jax import lax
from jax.experimental import pallas as pl
from jax.experimental.pallas import tpu as pltpu
from jax.experimental.pallas import tpu_sc as plsc

D = 128
NC, NS = 2, 16
NW = NC * NS
GE = 2
NBUF = 4


@functools.partial(jax.jit, static_argnames=("batch", "hist"))
def _lookup(idxp, table, *, batch, hist):
    gl = GE * hist
    glp = (gl + 15) // 16 * 16
    groups = batch // (NW * GE)
    epw = batch // NW
    mesh = plsc.VectorSubcoreMesh(core_axis_name="c", subcore_axis_name="s")

    @functools.partial(
        pl.kernel,
        out_type=jax.ShapeDtypeStruct((batch, hist, D), jnp.float32),
        mesh=mesh,
        scratch_types=[
            pltpu.VMEM((groups * glp,), jnp.int32),
            pltpu.VMEM((NBUF, glp, D), jnp.float32),
            pltpu.SemaphoreType.DMA((NBUF,)),
            pltpu.SemaphoreType.DMA((NBUF,)),
        ],
    )
    def body(table_hbm, idx_hbm, out_hbm, idx_v, rows_v, gsem, wsem):
        wid = lax.axis_index("s") * NC + lax.axis_index("c")
        pltpu.sync_copy(idx_hbm.at[pl.ds(wid * groups * glp, groups * glp)],
                        idx_v)
        ebase = wid * epw

        def fire_gather(g, b):
            pltpu.async_copy(table_hbm.at[idx_v.at[pl.ds(g * glp, glp)]],
                             rows_v.at[b], gsem.at[b])

        def wait_gather(b):
            pltpu.make_async_copy(table_hbm.at[idx_v.at[pl.ds(0, glp)]],
                                  rows_v.at[b], gsem.at[b]).wait()

        def fire_writes(g, b):
            for e in range(GE):
                pltpu.async_copy(rows_v.at[b].at[pl.ds(e * hist, hist)],
                                 out_hbm.at[ebase + g * GE + e], wsem.at[b])

        def wait_writes(b):
            for e in range(GE):
                pltpu.make_async_copy(rows_v.at[b].at[pl.ds(0, hist)],
                                      out_hbm.at[0], wsem.at[b]).wait()

        for b in range(NBUF):
            fire_gather(b, b)

        @pl.loop(0, groups - NBUF, step=NBUF)
        def _(g0):
            for b in range(NBUF):
                wait_gather(b)
                fire_writes(g0 + b, b)
            for b in range(NBUF):
                wait_writes(b)
                fire_gather(g0 + NBUF + b, b)

        for b in range(NBUF):
            wait_gather(b)
            fire_writes(groups - NBUF + b, b)
        for b in range(NBUF):
            wait_writes(b)

    return body(table, idxp)


def kernel(input_ids, word_embeddings):
    batch, hist = input_ids.shape
    gl = GE * hist
    glp = (gl + 15) // 16 * 16
    idx = input_ids.astype(jnp.int32).reshape(batch // GE, gl)
    idxp = jnp.pad(idx, ((0, 0), (0, glp - gl))).reshape(-1)
    return _lookup(idxp, word_embeddings, batch=batch, hist=hist)


# speedup vs baseline: 7.1864x; 7.1864x over previous
import functools

import jax
import jax.numpy as jnp
from jax import lax
from jax.experimental import pallas as pl
from jax.experimental.pallas import tpu as pltpu
from jax.experimental.pallas import tpu_sc as plsc

D = 128
NC, NS = 2, 16
NW = NC * NS
GE = 2
NBUF = 4


@functools.partial(jax.jit, static_argnames=("batch", "hist"))
def _lookup(idxp, table, *, batch, hist):
    gl = GE * hist
    glp = (gl + 15) // 16 * 16
    groups = batch // (NW * GE)
    epw = batch // NW
    mesh = plsc.VectorSubcoreMesh(core_axis_name="c", subcore_axis_name="s")

    @functools.partial(
        pl.kernel,
        out_type=jax.ShapeDtypeStruct((batch, hist, D), jnp.float32),
        mesh=mesh,
        scratch_types=[
            pltpu.VMEM((groups * glp,), jnp.int32),
            pltpu.VMEM((NBUF, glp, D), jnp.float32),
            pltpu.SemaphoreType.DMA((NBUF,)),
            pltpu.SemaphoreType.DMA((NBUF,)),
        ],
    )
    def body(table_hbm, idx_hbm, out_hbm, idx_v, rows_v, gsem, wsem):
        wid = lax.axis_index("s") * NC + lax.axis_index("c")
        pltpu.sync_copy(idx_hbm.at[pl.ds(wid * groups * glp, groups * glp)],
                        idx_v)
        ebase = wid * epw

        def fire_gather(g, b):
            pltpu.async_copy(table_hbm.at[idx_v.at[pl.ds(g * glp, glp)]],
                             rows_v.at[b], gsem.at[b])

        def wait_gather(b):
            pltpu.make_async_copy(table_hbm.at[idx_v.at[pl.ds(0, glp)]],
                                  rows_v.at[b], gsem.at[b]).wait()

        def fire_writes(g, b):
            for e in range(GE):
                pltpu.async_copy(rows_v.at[b].at[pl.ds(e * hist, hist)],
                                 out_hbm.at[ebase + g * GE + e], wsem.at[b])

        def wait_writes(b):
            for e in range(GE):
                pltpu.make_async_copy(rows_v.at[b].at[pl.ds(0, hist)],
                                      out_hbm.at[0], wsem.at[b]).wait()

        for b in range(NBUF):
            fire_gather(b, b)

        @pl.loop(0, groups - NBUF, step=NBUF)
        def _(g0):
            for b in range(NBUF):
                wait_gather(b)
                fire_writes(g0 + b, b)
            for b in range(NBUF):
                wait_writes(b)
                fire_gather(g0 + NBUF + b, b)

        for b in range(NBUF):
            wait_gather(b)
            fire_writes(groups - NBUF + b, b)
        for b in range(NBUF):
            wait_writes(b)

    return body(table, idxp)


def kernel(input_ids, word_embeddings):
    batch, hist = input_ids.shape
    table_rows = word_embeddings.shape[0]
    gl = GE * hist
    glp = (gl + 15) // 16 * 16
    idx = input_ids.astype(jnp.int32).reshape(batch // GE, gl)
    nrow = batch // GE
    pad = (jnp.arange(nrow * (glp - gl), dtype=jnp.int32) * 127) \
        % jnp.int32(table_rows)
    idxp = jnp.concatenate([idx, pad.reshape(nrow, glp - gl)], axis=1)
    idxp = idxp.reshape(-1)
    return _lookup(idxp, word_embeddings, batch=batch, hist=hist)


# speedup vs baseline: 7.3978x; 1.0294x over previous
import functools

import jax
import jax.numpy as jnp
from jax import lax
from jax.experimental import pallas as pl
from jax.experimental.pallas import tpu as pltpu
from jax.experimental.pallas import tpu_sc as plsc

D = 128
NC, NS = 2, 16
NW = NC * NS
GE = 2
NBUF = 4


@functools.partial(jax.jit, static_argnames=("batch", "hist"))
def _lookup(idxp, table, *, batch, hist):
    gl = GE * hist
    glp = (gl + 7) // 8 * 8
    groups = batch // (NW * GE)
    epw = batch // NW
    mesh = plsc.VectorSubcoreMesh(core_axis_name="c", subcore_axis_name="s")

    @functools.partial(
        pl.kernel,
        out_type=jax.ShapeDtypeStruct((batch, hist, D), jnp.float32),
        mesh=mesh,
        scratch_types=[
            pltpu.VMEM((groups * glp,), jnp.int32),
            pltpu.VMEM((NBUF, glp, D), jnp.float32),
            pltpu.SemaphoreType.DMA((NBUF,)),
            pltpu.SemaphoreType.DMA((NBUF,)),
        ],
    )
    def body(table_hbm, idx_hbm, out_hbm, idx_v, rows_v, gsem, wsem):
        wid = lax.axis_index("s") * NC + lax.axis_index("c")
        pltpu.sync_copy(idx_hbm.at[pl.ds(wid * groups * glp, groups * glp)],
                        idx_v)
        ebase = wid * epw

        def fire_gather(g, b):
            pltpu.async_copy(table_hbm.at[idx_v.at[pl.ds(g * glp, glp)]],
                             rows_v.at[b], gsem.at[b])

        def wait_gather(b):
            pltpu.make_async_copy(table_hbm.at[idx_v.at[pl.ds(0, glp)]],
                                  rows_v.at[b], gsem.at[b]).wait()

        def fire_writes(g, b):
            for e in range(GE):
                pltpu.async_copy(rows_v.at[b].at[pl.ds(e * hist, hist)],
                                 out_hbm.at[ebase + g * GE + e], wsem.at[b])

        def wait_writes(b):
            for e in range(GE):
                pltpu.make_async_copy(rows_v.at[b].at[pl.ds(0, hist)],
                                      out_hbm.at[0], wsem.at[b]).wait()

        for b in range(NBUF):
            fire_gather(b, b)

        @pl.loop(0, groups - NBUF, step=NBUF)
        def _(g0):
            for b in range(NBUF):
                wait_gather(b)
                fire_writes(g0 + b, b)
            for b in range(NBUF):
                wait_writes(b)
                fire_gather(g0 + NBUF + b, b)

        for b in range(NBUF):
            wait_gather(b)
            fire_writes(groups - NBUF + b, b)
        for b in range(NBUF):
            wait_writes(b)

    return body(table, idxp)


def kernel(input_ids, word_embeddings):
    batch, hist = input_ids.shape
    table_rows = word_embeddings.shape[0]
    gl = GE * hist
    glp = (gl + 7) // 8 * 8
    idx = input_ids.astype(jnp.int32).reshape(batch // GE, gl)
    nrow = batch // GE
    pad = (jnp.arange(nrow * (glp - gl), dtype=jnp.int32) * 127) \
        % jnp.int32(table_rows)
    idxp = jnp.concatenate([idx, pad.reshape(nrow, glp - gl)], axis=1)
    idxp = idxp.reshape(-1)
    return _lookup(idxp, word_embeddings, batch=batch, hist=hist)


# speedup vs baseline: 7.4108x; 1.0018x over previous
import functools

import jax
import jax.numpy as jnp
from jax import lax
from jax.experimental import pallas as pl
from jax.experimental.pallas import tpu as pltpu
from jax.experimental.pallas import tpu_sc as plsc

D = 128
NC, NS = 2, 16
NW = NC * NS
GE = 2
NBUF = 8


@functools.partial(jax.jit, static_argnames=("batch", "hist"))
def _lookup(idxp, table, *, batch, hist):
    gl = GE * hist
    glp = (gl + 7) // 8 * 8
    groups = batch // (NW * GE)
    epw = batch // NW
    mesh = plsc.VectorSubcoreMesh(core_axis_name="c", subcore_axis_name="s")

    @functools.partial(
        pl.kernel,
        out_type=jax.ShapeDtypeStruct((batch, hist, D), jnp.float32),
        mesh=mesh,
        scratch_types=[
            pltpu.VMEM((groups * glp,), jnp.int32),
            pltpu.VMEM((NBUF, glp, D), jnp.float32),
            pltpu.SemaphoreType.DMA((NBUF,)),
            pltpu.SemaphoreType.DMA((NBUF,)),
        ],
    )
    def body(table_hbm, idx_hbm, out_hbm, idx_v, rows_v, gsem, wsem):
        wid = lax.axis_index("s") * NC + lax.axis_index("c")
        pltpu.sync_copy(idx_hbm.at[pl.ds(wid * groups * glp, groups * glp)],
                        idx_v)
        ebase = wid * epw

        def fire_gather(g, b):
            pltpu.async_copy(table_hbm.at[idx_v.at[pl.ds(g * glp, glp)]],
                             rows_v.at[b], gsem.at[b])

        def wait_gather(b):
            pltpu.make_async_copy(table_hbm.at[idx_v.at[pl.ds(0, glp)]],
                                  rows_v.at[b], gsem.at[b]).wait()

        def fire_writes(g, b):
            for e in range(GE):
                pltpu.async_copy(rows_v.at[b].at[pl.ds(e * hist, hist)],
                                 out_hbm.at[ebase + g * GE + e], wsem.at[b])

        def wait_writes(b):
            for e in range(GE):
                pltpu.make_async_copy(rows_v.at[b].at[pl.ds(0, hist)],
                                      out_hbm.at[0], wsem.at[b]).wait()

        for b in range(NBUF):
            fire_gather(b, b)

        @pl.loop(0, groups - NBUF, step=NBUF)
        def _(g0):
            for b in range(NBUF):
                wait_gather(b)
                fire_writes(g0 + b, b)
            for b in range(NBUF):
                wait_writes(b)
                fire_gather(g0 + NBUF + b, b)

        for b in range(NBUF):
            wait_gather(b)
            fire_writes(groups - NBUF + b, b)
        for b in range(NBUF):
            wait_writes(b)

    return body(table, idxp)


def kernel(input_ids, word_embeddings):
    batch, hist = input_ids.shape
    table_rows = word_embeddings.shape[0]
    gl = GE * hist
    glp = (gl + 7) // 8 * 8
    idx = input_ids.astype(jnp.int32).reshape(batch // GE, gl)
    nrow = batch // GE
    pad = (jnp.arange(nrow * (glp - gl), dtype=jnp.int32) * 127) \
        % jnp.int32(table_rows)
    idxp = jnp.concatenate([idx, pad.reshape(nrow, glp - gl)], axis=1)
    idxp = idxp.reshape(-1)
    return _lookup(idxp, word_embeddings, batch=batch, hist=hist)


# speedup vs baseline: 13.2317x; 1.7854x over previous
import functools

import jax
import jax.numpy as jnp
from jax import lax
from jax.experimental import pallas as pl
from jax.experimental.pallas import tpu as pltpu
from jax.experimental.pallas import tpu_sc as plsc

D = 128
NC, NS = 2, 16
NW = NC * NS
CH = 128
NBUF = 5


@functools.partial(jax.jit, static_argnames=("batch", "hist"))
def _lookup(idx3, table, *, batch, hist):
    mesh = plsc.VectorSubcoreMesh(core_axis_name="c", subcore_axis_name="s")

    @functools.partial(
        pl.kernel,
        out_type=jax.ShapeDtypeStruct((hist, batch, D), jnp.float32),
        mesh=mesh,
        scratch_types=[
            pltpu.VMEM((hist, CH), jnp.int32),
            pltpu.VMEM((NBUF, CH, D), jnp.float32),
            pltpu.SemaphoreType.DMA((NBUF,)),
            pltpu.SemaphoreType.DMA((NBUF,)),
        ],
    )
    def body(table_hbm, idx_hbm, out_hbm, idx_v, rows_v, gsem, wsem):
        wid = lax.axis_index("s") * NC + lax.axis_index("c")
        pltpu.sync_copy(idx_hbm.at[wid], idx_v)
        bbase = wid * CH

        def fire_gather(h, b):
            pltpu.async_copy(table_hbm.at[idx_v.at[h]], rows_v.at[b],
                             gsem.at[b])

        def wait_gather(b):
            pltpu.make_async_copy(table_hbm.at[idx_v.at[0]], rows_v.at[b],
                                  gsem.at[b]).wait()

        def fire_write(h, b):
            pltpu.async_copy(rows_v.at[b],
                             out_hbm.at[h].at[pl.ds(bbase, CH)], wsem.at[b])

        def wait_write(b):
            pltpu.make_async_copy(rows_v.at[b],
                                  out_hbm.at[0].at[pl.ds(bbase, CH)],
                                  wsem.at[b]).wait()

        for b in range(NBUF):
            fire_gather(b, b)

        @pl.loop(0, hist - NBUF, step=NBUF)
        def _(h0):
            for b in range(NBUF):
                wait_gather(b)
                fire_write(h0 + b, b)
            for b in range(NBUF):
                wait_write(b)
                fire_gather(h0 + NBUF + b, b)

        for b in range(NBUF):
            wait_gather(b)
            fire_write(hist - NBUF + b, b)
        for b in range(NBUF):
            wait_write(b)

    return body(table, idx3)


def kernel(input_ids, word_embeddings):
    batch, hist = input_ids.shape
    idx3 = input_ids.astype(jnp.int32).reshape(NW, CH, hist)
    idx3 = idx3.transpose(0, 2, 1)
    out = _lookup(idx3, word_embeddings, batch=batch, hist=hist)
    return out.transpose(1, 0, 2)


# speedup vs baseline: 13.5212x; 1.0219x over previous
import functools

import jax
import jax.numpy as jnp
from jax import lax
from jax.experimental import pallas as pl
from jax.experimental.pallas import tpu as pltpu
from jax.experimental.pallas import tpu_sc as plsc

D = 128
NC, NS = 2, 16
NW = NC * NS
CH = 128
HCH = 64
NBUF = 10


@functools.partial(jax.jit, static_argnames=("batch", "hist"))
def _lookup(idx3, table, *, batch, hist):
    mesh = plsc.VectorSubcoreMesh(core_axis_name="c", subcore_axis_name="s")

    @functools.partial(
        pl.kernel,
        out_type=jax.ShapeDtypeStruct((hist, batch, D), jnp.float32),
        mesh=mesh,
        scratch_types=[
            pltpu.VMEM((hist, CH), jnp.int32),
            pltpu.VMEM((NBUF, HCH, D), jnp.float32),
            pltpu.SemaphoreType.DMA((NBUF,)),
            pltpu.SemaphoreType.DMA((NBUF,)),
        ],
    )
    def body(table_hbm, idx_hbm, out_hbm, idx_v, rows_v, gsem, wsem):
        wid = lax.axis_index("s") * NC + lax.axis_index("c")
        pltpu.sync_copy(idx_hbm.at[wid], idx_v)
        bbase = wid * CH

        def fire_gather(h, half, b):
            pltpu.async_copy(
                table_hbm.at[idx_v.at[h].at[pl.ds(half * HCH, HCH)]],
                rows_v.at[b], gsem.at[b])

        def wait_gather(b):
            pltpu.make_async_copy(
                table_hbm.at[idx_v.at[0].at[pl.ds(0, HCH)]], rows_v.at[b],
                gsem.at[b]).wait()

        def fire_write(h, half, b):
            pltpu.async_copy(
                rows_v.at[b],
                out_hbm.at[h].at[pl.ds(bbase + half * HCH, HCH)], wsem.at[b])

        def wait_write(b):
            pltpu.make_async_copy(rows_v.at[b],
                                  out_hbm.at[0].at[pl.ds(bbase, HCH)],
                                  wsem.at[b]).wait()

        hb = NBUF // 2

        for j in range(NBUF):
            fire_gather(j // 2, j % 2, j)

        @pl.loop(0, hist - hb, step=hb)
        def _(h0):
            for j in range(NBUF):
                wait_gather(j)
                fire_write(h0 + j // 2, j % 2, j)
            for j in range(NBUF):
                wait_write(j)
                fire_gather(h0 + hb + j // 2, j % 2, j)

        for j in range(NBUF):
            wait_gather(j)
            fire_write(hist - hb + j // 2, j % 2, j)
        for j in range(NBUF):
            wait_write(j)

    return body(table, idx3)


def kernel(input_ids, word_embeddings):
    batch, hist = input_ids.shape
    idx3 = input_ids.astype(jnp.int32).reshape(NW, CH, hist)
    idx3 = idx3.transpose(0, 2, 1)
    out = _lookup(idx3, word_embeddings, batch=batch, hist=hist)
    return out.transpose(1, 0, 2)
